# Initial kernel scaffold; baseline (speedup 1.0000x reference)
import jax
import jax.numpy as jnp
from jax.experimental import pallas as pl


def kernel(frac, pe):
    raise NotImplementedError("write your pallas kernel here")


# speedup vs baseline: 2.1576x; 2.1576x over previous
import functools

import jax
import jax.numpy as jnp
from jax import lax
from jax.experimental import pallas as pl
from jax.experimental.pallas import tpu as pltpu
from jax.experimental.pallas import tpu_sc as plsc

RES = 100
D = 256
LANES = 16
MAGIC = 12582912.0
K = 128


def _encoder_kernel(n_rows, n_workers):
    per_w = n_rows // n_workers
    n_chunks = per_w // K
    mesh = plsc.VectorSubcoreMesh(core_axis_name="c", subcore_axis_name="s")

    @functools.partial(
        pl.kernel,
        mesh=mesh,
        out_type=jax.ShapeDtypeStruct((n_rows, D), jnp.float32),
        scratch_types=[
            pltpu.VMEM((K,), jnp.float32),
            pltpu.VMEM((K,), jnp.int32),
            pltpu.VMEM((K, D), jnp.float32),
            pltpu.SemaphoreType.DMA,
        ],
    )
    def body(frac_hbm, pe_hbm, out_hbm, frac_v, idx_v, rows_v, sem):
        wid = lax.axis_index("s") * 2 + lax.axis_index("c")
        base = wid * per_w

        def step(g, carry):
            off = base + g * K
            pltpu.sync_copy(frac_hbm.at[pl.ds(off, K)], frac_v)
            for j in range(K // LANES):
                v = frac_v[pl.ds(j * LANES, LANES)]
                t = jnp.maximum(v, jnp.float32(1.0 / RES)) * jnp.float32(RES)
                r = (t + jnp.float32(MAGIC)) - jnp.float32(MAGIC)
                idx_v[pl.ds(j * LANES, LANES)] = r.astype(jnp.int32) - 1
            pltpu.async_copy(pe_hbm.at[idx_v], rows_v, sem).wait()
            pltpu.sync_copy(rows_v, out_hbm.at[pl.ds(off, K)])
            return carry

        lax.fori_loop(0, n_chunks, step, 0)

    return body


def kernel(frac, pe):
    b, s = frac.shape
    n_rows = b * s
    out = _encoder_kernel(n_rows, 32)(frac.reshape(n_rows), pe)
    return out.reshape(b, s, D)
